# Initial kernel scaffold; baseline (speedup 1.0000x reference)
import jax
import jax.numpy as jnp
from jax.experimental import pallas as pl


def kernel(x, table):
    raise NotImplementedError("write your pallas kernel here")


# speedup vs baseline: 1.8291x; 1.8291x over previous
import jax
import jax.numpy as jnp
from jax.experimental import pallas as pl
from jax.experimental.pallas import tpu as pltpu

_L = 512
_D = 128
_WIN_LO = _D * _D - 1 - (_L - 1)


def _add_kernel(win_ref, x_ref, o_ref):
    i = pl.program_id(1)
    shifted = win_ref[pl.ds(_L - 1 - i, _L), :]
    o_ref[...] = x_ref[...] + shifted[None, None, :, :]


def kernel(x, table):
    win = jax.lax.slice(table, (_WIN_LO, 0), (_WIN_LO + 2 * _L, _D))
    grid = (x.shape[0], _L)
    return pl.pallas_call(
        _add_kernel,
        grid=grid,
        in_specs=[
            pl.BlockSpec((2 * _L, _D), lambda b, i: (0, 0)),
            pl.BlockSpec((1, 1, _L, _D), lambda b, i: (b, i, 0, 0)),
        ],
        out_specs=pl.BlockSpec((1, 1, _L, _D), lambda b, i: (b, i, 0, 0)),
        out_shape=jax.ShapeDtypeStruct(x.shape, x.dtype),
    )(win, x)


# speedup vs baseline: 6.1051x; 3.3378x over previous
import jax
import jax.numpy as jnp
from jax.experimental import pallas as pl
from jax.experimental.pallas import tpu as pltpu

_L = 512
_D = 128
_WIN_LO = _D * _D - 1 - (_L - 1)


_BI = 8


def _add_kernel(win_ref, x_ref, o_ref):
    ib = pl.program_id(1)
    base = _L - 1 - ib * _BI
    for li in range(_BI):
        shifted = win_ref[pl.ds(base - li, _L), :]
        o_ref[0, li] = x_ref[0, li] + shifted


def kernel(x, table):
    win = jax.lax.slice(table, (_WIN_LO, 0), (_WIN_LO + 2 * _L, _D))
    grid = (x.shape[0], _L // _BI)
    return pl.pallas_call(
        _add_kernel,
        grid=grid,
        in_specs=[
            pl.BlockSpec((2 * _L, _D), lambda b, i: (0, 0)),
            pl.BlockSpec((1, _BI, _L, _D), lambda b, i: (b, i, 0, 0)),
        ],
        out_specs=pl.BlockSpec((1, _BI, _L, _D), lambda b, i: (b, i, 0, 0)),
        out_shape=jax.ShapeDtypeStruct(x.shape, x.dtype),
        compiler_params=pltpu.CompilerParams(
            dimension_semantics=("parallel", "parallel"),
        ),
    )(win, x)


# speedup vs baseline: 6.7758x; 1.1099x over previous
import jax
import jax.numpy as jnp
from jax.experimental import pallas as pl
from jax.experimental.pallas import tpu as pltpu

_L = 512
_D = 128
_WIN_LO = _D * _D - 1 - (_L - 1)


_BI = 16


def _add_kernel(win_ref, x_ref, o_ref):
    ib = pl.program_id(1)
    base = _L - 1 - ib * _BI
    for li in range(_BI):
        shifted = win_ref[pl.ds(base - li, _L), :]
        o_ref[0, li] = x_ref[0, li] + shifted


def kernel(x, table):
    win = jax.lax.slice(table, (_WIN_LO, 0), (_WIN_LO + 2 * _L, _D))
    grid = (x.shape[0], _L // _BI)
    return pl.pallas_call(
        _add_kernel,
        grid=grid,
        in_specs=[
            pl.BlockSpec((2 * _L, _D), lambda b, i: (0, 0)),
            pl.BlockSpec((1, _BI, _L, _D), lambda b, i: (b, i, 0, 0)),
        ],
        out_specs=pl.BlockSpec((1, _BI, _L, _D), lambda b, i: (b, i, 0, 0)),
        out_shape=jax.ShapeDtypeStruct(x.shape, x.dtype),
        compiler_params=pltpu.CompilerParams(
            dimension_semantics=("parallel", "parallel"),
        ),
    )(win, x)


# speedup vs baseline: 6.8875x; 1.0165x over previous
import jax
import jax.numpy as jnp
from jax.experimental import pallas as pl
from jax.experimental.pallas import tpu as pltpu

_L = 512
_D = 128
_WIN_LO = _D * _D - 1 - (_L - 1)


_BI = 32


def _add_kernel(win_ref, x_ref, o_ref):
    ib = pl.program_id(1)
    base = _L - 1 - ib * _BI
    for li in range(_BI):
        shifted = win_ref[pl.ds(base - li, _L), :]
        o_ref[0, li] = x_ref[0, li] + shifted


def kernel(x, table):
    win = jax.lax.slice(table, (_WIN_LO, 0), (_WIN_LO + 2 * _L, _D))
    grid = (x.shape[0], _L // _BI)
    return pl.pallas_call(
        _add_kernel,
        grid=grid,
        in_specs=[
            pl.BlockSpec((2 * _L, _D), lambda b, i: (0, 0)),
            pl.BlockSpec((1, _BI, _L, _D), lambda b, i: (b, i, 0, 0)),
        ],
        out_specs=pl.BlockSpec((1, _BI, _L, _D), lambda b, i: (b, i, 0, 0)),
        out_shape=jax.ShapeDtypeStruct(x.shape, x.dtype),
        compiler_params=pltpu.CompilerParams(
            dimension_semantics=("parallel", "parallel"),
        ),
    )(win, x)
